# baseline (speedup 1.0000x reference)
import jax
import jax.numpy as jnp
from jax import lax
from jax.experimental import pallas as pl
from jax.experimental.pallas import tpu as pltpu
from jax.experimental.pallas import tpu_sc as plsc

N = 10000
NP = 10240
D = 256
DH = 128
NNZ = 160000
NNZ_P = NNZ
R_OPS = 3
THRESH = 0.0001

NTILES = 16
EPT = NNZ_P // NTILES
E = 80
NCHUNK = EPT // E
ROWS_PT = NP // NTILES
CH = 80
NV = DH // 16


def _mm_body(x_ref, w_ref, o_ref):
    o_ref[...] = jnp.dot(x_ref[...], w_ref[...],
                         preferred_element_type=jnp.float32)


def _matmul_split(x, weight):
    return pl.pallas_call(
        _mm_body,
        grid=(2, 125),
        in_specs=[
            pl.BlockSpec((80, D), lambda j, i: (i, 0)),
            pl.BlockSpec((D, DH), lambda j, i: (0, j)),
        ],
        out_specs=pl.BlockSpec((80, DH), lambda j, i: (j * 128 + i, 0)),
        out_shape=jax.ShapeDtypeStruct((2 * NP, DH), jnp.float32),
    )(x, weight)


def _sc_body(x1s, rows3, cols3, vals3, filt3,
             x5s, zbuf,
             acc,
             g0, g1, r0, r1, v0, v1,
             cbig, fbuf, sg0, sg1, ss0, ss1):
    cid = lax.axis_index("c")
    sid = lax.axis_index("s")
    edge0 = sid * EPT
    row0 = sid * ROWS_PT
    zvec = jnp.zeros((16,), jnp.float32)
    gb = [g0, g1]
    rb = [r0, r1]
    vb = [v0, v1]
    sg = [sg0, sg1]
    ss = [ss0, ss1]

    def zero_acc():
        def zb(i, _):
            for q in range(NV):
                g0[i, pl.ds(q * 16, 16)] = zvec
            return 0
        lax.fori_loop(0, CH, zb, 0)
        for k in range(ROWS_PT // CH):
            pltpu.sync_copy(g0, acc.at[pl.ds(row0 + k * CH, CH)])

    def scatter_round(table, r, base):
        eoff = r * NNZ_P + edge0
        pltpu.sync_copy(cols3.at[pl.ds(eoff, EPT)], cbig)

        def addb(i, _):
            cbig[pl.ds(i * 16, 16)] = cbig[pl.ds(i * 16, 16)] + base
            return 0
        lax.fori_loop(0, EPT // 16, addb, 0)

        def issue(ci, k):
            off = ci * E
            pltpu.async_copy(rows3.at[pl.ds(eoff + off, E)], rb[k], sg[k])
            pltpu.async_copy(vals3.at[pl.ds(eoff + off, E)], vb[k], sg[k])
            pltpu.async_copy(table.at[cbig.at[pl.ds(off, E)]], gb[k],
                             sg[k])

        def wait_gather(ci, k):
            off = ci * E
            pltpu.make_async_copy(rows3.at[pl.ds(eoff + off, E)], rb[k],
                                  sg[k]).wait()
            pltpu.make_async_copy(vals3.at[pl.ds(eoff + off, E)], vb[k],
                                  sg[k]).wait()
            pltpu.make_async_copy(table.at[cbig.at[pl.ds(off, E)]], gb[k],
                                  sg[k]).wait()

        def scale(k):
            gbk, vbk = gb[k], vb[k]
            def grp(g, _):
                v16 = vbk[pl.ds(g * 16, 16)]
                for j in range(16):
                    vv = jnp.full((16,), v16[j], jnp.float32)
                    e = g * 16 + j
                    for q in range(NV):
                        gbk[e, pl.ds(q * 16, 16)] = \
                            gbk[e, pl.ds(q * 16, 16)] * vv
                return 0
            lax.fori_loop(0, E // 16, grp, 0)

        def fire_scatter(k):
            pltpu.async_copy(gb[k], acc.at[rb[k]], ss[k], add=True)

        def wait_scatter(k):
            pltpu.make_async_copy(gb[k], acc.at[rb[k]], ss[k]).wait()

        issue(0, 0)
        issue(1, 1)
        wait_gather(0, 0)
        scale(0)
        fire_scatter(0)
        wait_gather(1, 1)
        scale(1)
        wait_scatter(0)
        issue(2, 0)
        fire_scatter(1)
        def dchunk(g, _):
            c0 = 2 * g
            wait_scatter(1)
            issue(c0 + 1, 1)
            wait_gather(c0, 0)
            scale(0)
            fire_scatter(0)
            wait_gather(c0 + 1, 1)
            scale(1)
            wait_scatter(0)
            issue(c0 + 2, 0)
            fire_scatter(1)
            return 0
        lax.fori_loop(1, (NCHUNK - 1) // 2, dchunk, 0)
        wait_scatter(1)
        wait_gather(NCHUNK - 1, 0)
        scale(0)
        fire_scatter(0)
        wait_scatter(0)

    def round_body(r, _):
        zero_acc()
        plsc.subcore_barrier()
        scatter_round(x1s, r, cid * NP)
        plsc.subcore_barrier()
        pltpu.sync_copy(filt3.at[pl.ds(r * NP + row0, ROWS_PT)], fbuf)
        zbase = cid * (R_OPS * NP) + r * NP + row0
        for k in range(ROWS_PT // CH):
            pltpu.sync_copy(acc.at[pl.ds(row0 + k * CH, CH)], g0)
            def rowfn(g, _):
                f16 = fbuf[pl.ds(k * CH + g * 16, 16)]
                for j in range(16):
                    fv = jnp.full((16,), f16[j], jnp.float32)
                    i = g * 16 + j
                    for q in range(NV):
                        y = g0[i, pl.ds(q * 16, 16)]
                        s = jnp.sign(y) * jnp.maximum(jnp.abs(y) - THRESH,
                                                      0.0)
                        g0[i, pl.ds(q * 16, 16)] = s * fv
                return 0
            lax.fori_loop(0, CH // 16, rowfn, 0)
            pltpu.sync_copy(g0, zbuf.at[pl.ds(zbase + k * CH, CH)])
        plsc.subcore_barrier()
        return 0

    lax.fori_loop(0, R_OPS, round_body, 0)

    zero_acc()
    plsc.subcore_barrier()
    def round2(r, _):
        scatter_round(zbuf, r, cid * (R_OPS * NP) + r * NP)
        return 0
    lax.fori_loop(0, R_OPS, round2, 0)
    plsc.subcore_barrier()
    for k in range(ROWS_PT // CH):
        pltpu.sync_copy(acc.at[pl.ds(row0 + k * CH, CH)], g0)
        pltpu.sync_copy(g0, x5s.at[pl.ds(cid * NP + row0 + k * CH, CH)])


def _sc_spmm(x1s, rows3, cols3, vals3, filt3):
    mesh = plsc.VectorSubcoreMesh(core_axis_name="c", subcore_axis_name="s")
    f = pl.kernel(
        _sc_body,
        out_type=[
            jax.ShapeDtypeStruct((2 * NP, DH), jnp.float32),
            jax.ShapeDtypeStruct((2 * R_OPS * NP, DH), jnp.float32),
        ],
        mesh=mesh,
        scratch_types=(
            [pltpu.VMEM_SHARED((NP, DH), jnp.float32)]
            + [pltpu.VMEM((E, DH), jnp.float32) for _ in range(2)]
            + [pltpu.VMEM((E,), jnp.int32) for _ in range(2)]
            + [pltpu.VMEM((E,), jnp.float32) for _ in range(2)]
            + [pltpu.VMEM((EPT,), jnp.int32),
               pltpu.VMEM((ROWS_PT,), jnp.float32)]
            + [pltpu.SemaphoreType.DMA for _ in range(4)]
        ),
    )
    return f(x1s, rows3, cols3, vals3, filt3)


def _fin_body(a_ref, x_ref, x5_ref, b_ref, o_ref):
    av = a_ref[0]
    o_ref[...] = av * x_ref[...] + (1.0 - av) * (x5_ref[...] + b_ref[0])


def _finalize(x, x5s, bias, a):
    af = jnp.asarray(a, jnp.float32).reshape(1)
    bias2 = bias.reshape(2, 1, DH)
    return pl.pallas_call(
        _fin_body,
        grid=(125, 2),
        in_specs=[
            pl.BlockSpec(memory_space=pltpu.SMEM),
            pl.BlockSpec((80, DH), lambda i, j: (i, j)),
            pl.BlockSpec((80, DH), lambda i, j: (j * 128 + i, 0)),
            pl.BlockSpec((1, 1, DH), lambda i, j: (j, 0, 0)),
        ],
        out_specs=pl.BlockSpec((80, DH), lambda i, j: (i, j)),
        out_shape=jax.ShapeDtypeStruct((N, D), jnp.float32),
    )(af, x, x5s, bias2)


def kernel(x, rows, cols, vals, weight, filt, bias, a):
    x1s = _matmul_split(x, weight)
    rows3 = rows[1:].reshape(-1)
    cols3 = cols[1:].reshape(-1)
    vals3 = vals[1:].reshape(-1)
    filt3 = jnp.pad(filt[N:, 0].reshape(R_OPS, N),
                    ((0, 0), (0, NP - N))).reshape(-1)
    x5s, _ = _sc_spmm(x1s, rows3, cols3, vals3, filt3)
    return _finalize(x, x5s, bias, a)


# speedup vs baseline: 1.0438x; 1.0438x over previous
import jax
import jax.numpy as jnp
from jax import lax
from jax.experimental import pallas as pl
from jax.experimental.pallas import tpu as pltpu
from jax.experimental.pallas import tpu_sc as plsc

N = 10000
NP = 10240
D = 256
DH = 128
NNZ = 160000
NNZ_P = NNZ
R_OPS = 3
THRESH = 0.0001

NTILES = 16
EPT = NNZ_P // NTILES
E = 80
NCHUNK = EPT // E
ROWS_PT = NP // NTILES
CH = 80
NV = DH // 16


def _mm_body(x_ref, w_ref, o_ref):
    o_ref[...] = jnp.dot(x_ref[...], w_ref[...],
                         preferred_element_type=jnp.float32)


def _matmul_split(x, weight):
    return pl.pallas_call(
        _mm_body,
        grid=(2, 125),
        in_specs=[
            pl.BlockSpec((80, D), lambda j, i: (i, 0)),
            pl.BlockSpec((D, DH), lambda j, i: (0, j)),
        ],
        out_specs=pl.BlockSpec((80, DH), lambda j, i: (j * 128 + i, 0)),
        out_shape=jax.ShapeDtypeStruct((2 * NP, DH), jnp.float32),
    )(x, weight)


def _sc_body(x1s, rows3, cols3, vals3, filt3,
             x5s, zbuf,
             acc,
             g0, g1, r0, r1, v0, v1,
             cbig, fbuf, sg0, sg1):
    cid = lax.axis_index("c")
    sid = lax.axis_index("s")
    edge0 = sid * EPT
    row0 = sid * ROWS_PT
    zvec = jnp.zeros((16,), jnp.float32)
    gb = [g0, g1]
    rb = [r0, r1]
    vb = [v0, v1]
    sg = [sg0, sg1]

    def zero_acc():
        def zb(i, _):
            for q in range(NV):
                g0[i, pl.ds(q * 16, 16)] = zvec
            return 0
        lax.fori_loop(0, CH, zb, 0)
        for k in range(ROWS_PT // CH):
            pltpu.sync_copy(g0, acc.at[pl.ds(row0 + k * CH, CH)])

    def scatter_round(table, r, base):
        eoff = r * NNZ_P + edge0
        pltpu.sync_copy(cols3.at[pl.ds(eoff, EPT)], cbig)

        def addb(i, _):
            cbig[pl.ds(i * 16, 16)] = cbig[pl.ds(i * 16, 16)] + base
            return 0
        lax.fori_loop(0, EPT // 16, addb, 0)

        def issue(ci, k):
            off = ci * E
            pltpu.async_copy(rows3.at[pl.ds(eoff + off, E)], rb[k], sg[k])
            pltpu.async_copy(vals3.at[pl.ds(eoff + off, E)], vb[k], sg[k])
            pltpu.async_copy(table.at[cbig.at[pl.ds(off, E)]], gb[k],
                             sg[k])

        def wait_gather(ci, k):
            off = ci * E
            pltpu.make_async_copy(rows3.at[pl.ds(eoff + off, E)], rb[k],
                                  sg[k]).wait()
            pltpu.make_async_copy(vals3.at[pl.ds(eoff + off, E)], vb[k],
                                  sg[k]).wait()
            pltpu.make_async_copy(table.at[cbig.at[pl.ds(off, E)]], gb[k],
                                  sg[k]).wait()

        def scale(k):
            gbk, vbk = gb[k], vb[k]
            def grp(g, _):
                v16 = vbk[pl.ds(g * 16, 16)]
                for j in range(16):
                    vv = jnp.full((16,), v16[j], jnp.float32)
                    e = g * 16 + j
                    for q in range(NV):
                        gbk[e, pl.ds(q * 16, 16)] = \
                            gbk[e, pl.ds(q * 16, 16)] * vv
                return 0
            lax.fori_loop(0, E // 16, grp, 0)

        def finish(ci, k):
            wait_gather(ci, k)
            scale(k)
            pltpu.sync_copy(gb[k], acc.at[rb[k]], add=True)

        issue(0, 0)
        def dchunk(g, _):
            issue(2 * g + 1, 1)
            finish(2 * g, 0)
            issue(2 * g + 2, 0)
            finish(2 * g + 1, 1)
            return 0
        lax.fori_loop(0, (NCHUNK - 1) // 2, dchunk, 0)
        finish(NCHUNK - 1, 0)

    def round_body(r, _):
        zero_acc()
        plsc.subcore_barrier()
        scatter_round(x1s, r, cid * NP)
        plsc.subcore_barrier()
        pltpu.sync_copy(filt3.at[pl.ds(r * NP + row0, ROWS_PT)], fbuf)
        zbase = cid * (R_OPS * NP) + r * NP + row0
        for k in range(ROWS_PT // CH):
            pltpu.sync_copy(acc.at[pl.ds(row0 + k * CH, CH)], g0)
            def rowfn(g, _):
                f16 = fbuf[pl.ds(k * CH + g * 16, 16)]
                for j in range(16):
                    fv = jnp.full((16,), f16[j], jnp.float32)
                    i = g * 16 + j
                    for q in range(NV):
                        y = g0[i, pl.ds(q * 16, 16)]
                        s = jnp.sign(y) * jnp.maximum(jnp.abs(y) - THRESH,
                                                      0.0)
                        g0[i, pl.ds(q * 16, 16)] = s * fv
                return 0
            lax.fori_loop(0, CH // 16, rowfn, 0)
            pltpu.sync_copy(g0, zbuf.at[pl.ds(zbase + k * CH, CH)])
        plsc.subcore_barrier()
        return 0

    lax.fori_loop(0, R_OPS, round_body, 0)

    zero_acc()
    plsc.subcore_barrier()
    def round2(r, _):
        scatter_round(zbuf, r, cid * (R_OPS * NP) + r * NP)
        return 0
    lax.fori_loop(0, R_OPS, round2, 0)
    plsc.subcore_barrier()
    for k in range(ROWS_PT // CH):
        pltpu.sync_copy(acc.at[pl.ds(row0 + k * CH, CH)], g0)
        pltpu.sync_copy(g0, x5s.at[pl.ds(cid * NP + row0 + k * CH, CH)])


def _sc_spmm(x1s, rows3, cols3, vals3, filt3):
    mesh = plsc.VectorSubcoreMesh(core_axis_name="c", subcore_axis_name="s")
    f = pl.kernel(
        _sc_body,
        out_type=[
            jax.ShapeDtypeStruct((2 * NP, DH), jnp.float32),
            jax.ShapeDtypeStruct((2 * R_OPS * NP, DH), jnp.float32),
        ],
        mesh=mesh,
        scratch_types=(
            [pltpu.VMEM_SHARED((NP, DH), jnp.float32)]
            + [pltpu.VMEM((E, DH), jnp.float32) for _ in range(2)]
            + [pltpu.VMEM((E,), jnp.int32) for _ in range(2)]
            + [pltpu.VMEM((E,), jnp.float32) for _ in range(2)]
            + [pltpu.VMEM((EPT,), jnp.int32),
               pltpu.VMEM((ROWS_PT,), jnp.float32)]
            + [pltpu.SemaphoreType.DMA for _ in range(2)]
        ),
    )
    return f(x1s, rows3, cols3, vals3, filt3)


def _fin_body(a_ref, x_ref, x5_ref, b_ref, o_ref):
    av = a_ref[0]
    o_ref[...] = av * x_ref[...] + (1.0 - av) * (x5_ref[...] + b_ref[0])


def _finalize(x, x5s, bias, a):
    af = jnp.asarray(a, jnp.float32).reshape(1)
    bias2 = bias.reshape(2, 1, DH)
    return pl.pallas_call(
        _fin_body,
        grid=(125, 2),
        in_specs=[
            pl.BlockSpec(memory_space=pltpu.SMEM),
            pl.BlockSpec((80, DH), lambda i, j: (i, j)),
            pl.BlockSpec((80, DH), lambda i, j: (j * 128 + i, 0)),
            pl.BlockSpec((1, 1, DH), lambda i, j: (j, 0, 0)),
        ],
        out_specs=pl.BlockSpec((80, DH), lambda i, j: (i, j)),
        out_shape=jax.ShapeDtypeStruct((N, D), jnp.float32),
    )(af, x, x5s, bias2)


def kernel(x, rows, cols, vals, weight, filt, bias, a):
    x1s = _matmul_split(x, weight)
    rows3 = rows[1:].reshape(-1)
    cols3 = cols[1:].reshape(-1)
    vals3 = vals[1:].reshape(-1)
    filt3 = jnp.pad(filt[N:, 0].reshape(R_OPS, N),
                    ((0, 0), (0, NP - N))).reshape(-1)
    x5s, _ = _sc_spmm(x1s, rows3, cols3, vals3, filt3)
    return _finalize(x, x5s, bias, a)
